# baseline (speedup 1.0000x reference)
import functools

import jax
import jax.numpy as jnp
from jax import lax
from jax.experimental import pallas as pl
from jax.experimental.pallas import tpu as pltpu
from jax.experimental.pallas import tpu_sc as plsc

B, N, C, K = 8, 2000, 128, 9
NDS = 1000
EPS = 1e-5
NPAD = 2048


def _bn3(x):
    m = jnp.mean(x, axis=(0, 2), keepdims=True)
    v = jnp.var(x, axis=(0, 2), keepdims=True)
    return (x - m) / jnp.sqrt(v + EPS)


def _bn4(x):
    m = jnp.mean(x, axis=(0, 2, 3), keepdims=True)
    v = jnp.var(x, axis=(0, 2, 3), keepdims=True)
    return (x - m) / jnp.sqrt(v + EPS)


def _bn1(x):
    m = jnp.mean(x, axis=0, keepdims=True)
    v = jnp.var(x, axis=0, keepdims=True)
    return (x - m) / jnp.sqrt(v + EPS)


def _in3(x):
    m = jnp.mean(x, axis=2, keepdims=True)
    v = jnp.var(x, axis=2, keepdims=True)
    return (x - m) / jnp.sqrt(v + EPS)


def _c1(x, W, b):
    return jnp.einsum('oc,bcn->bon', W, x) + b[None, :, None]


def _c2(x, W, b, sw):
    out = lax.conv_general_dilated(
        x, W, window_strides=(1, sw), padding='VALID',
        dimension_numbers=('NCHW', 'OIHW', 'NCHW'))
    return out + b[None, :, None, None]


def _resnet(x, Wr, br, W1, b1, W2, b2, pre):
    x1 = _c1(x, Wr, br) if pre else x
    out = jax.nn.relu(_bn3(_in3(_c1(x, W1, b1))))
    out = _bn3(_in3(_c1(out, W2, b2)))
    return jax.nn.relu(out + x1)


def _graph_feature(x, k):
    inner = -2.0 * jnp.einsum('bcn,bcm->bnm', x, x)
    xx = jnp.sum(x * x, axis=1)
    pd = -xx[:, :, None] - inner - xx[:, None, :]
    _, idx = lax.top_k(pd, k)
    xt = jnp.transpose(x, (0, 2, 1))
    feat = jax.vmap(lambda a, i: a[i])(xt, idx)
    xrep = jnp.broadcast_to(xt[:, :, None, :], feat.shape)
    f = jnp.concatenate([xrep, xrep - feat], axis=3)
    return jnp.transpose(f, (0, 3, 1, 2))


def _scores(x, y, Wc, bc, Wr, br, Wl1, bl1, Wl2, bl2, Wbe, bbe, Wknn, bknn,
            Wg1, bg1, Wg2, bg2, Wd1, bd1, Wd2, bd2, W2l1, b2l1, W2l2, b2l2,
            Wlin, blin):
    xt = jnp.transpose(x[:, 0, :, :], (0, 2, 1))
    out0 = jax.nn.relu(_bn3(_c1(xt, Wc, bc)))
    d = _resnet(out0, Wr, br, Wl1, bl1, Wl2, bl2, True)
    x1 = jax.nn.relu(_bn3(_c1(d, Wbe, bbe)))
    xl = jax.nn.relu(_bn3(_c1(x1, Wknn, bknn)))
    f = _graph_feature(xl, K)
    h = jax.nn.relu(_bn4(_c2(f, Wd1, bd1, 3)))
    h = jax.nn.relu(_bn4(_c2(h, Wd2, bd2, 1)))
    gv = jnp.mean(x1, axis=2)
    g1 = jax.nn.relu(_bn1(gv @ Wg1.T + bg1))
    g2 = _bn1(g1 @ Wg2.T + bg2)
    xlg = g2[:, :, None, None] + h
    wei = jax.nn.sigmoid(xlg)[:, :, :, 0]
    out = 2.0 * d * wei + 2.0 * out0 * (1.0 - wei)
    out = _resnet(out, Wr, br, W2l1, b2l1, W2l2, b2l2, False)
    return _c1(out, Wlin, blin)[:, 0, :]


def _xor_swap(a, j):
    lower = (lax.broadcasted_iota(jnp.int32, a.shape, 1) & j) == 0
    return jnp.where(lower, jnp.roll(a, -j, axis=1), jnp.roll(a, j, axis=1))


def _topk_body(w_ref, idx_ref):
    v = w_ref[...]
    iota = lax.broadcasted_iota(jnp.int32, (B, NPAD), 1)
    idx = iota
    k = 2
    while k <= NPAD:
        j = k // 2
        while j >= 1:
            pv = _xor_swap(v, j)
            pidx = _xor_swap(idx, j)
            desc = (iota & k) == 0
            lower = (iota & j) == 0
            before = (v > pv) | ((v == pv) & (idx < pidx))
            take_self = desc == (lower == before)
            v = jnp.where(take_self, v, pv)
            idx = jnp.where(take_self, idx, pidx)
            j //= 2
        k *= 2
    idx_ref[...] = idx


def _topk_indices(w0):
    wp = jnp.concatenate(
        [w0, jnp.full((B, NPAD - N), -jnp.inf, jnp.float32)], axis=1)
    return pl.pallas_call(
        _topk_body,
        out_shape=jax.ShapeDtypeStruct((B, NPAD), jnp.int32),
    )(wp)


_D = 128
_NOUT = 8192
_NW = 32
_BPW = _NOUT // _NW


def _sc_gather(table, fidx):
    mesh = plsc.VectorSubcoreMesh(core_axis_name="c", subcore_axis_name="s")

    @functools.partial(
        pl.kernel, mesh=mesh,
        out_type=jax.ShapeDtypeStruct((_NOUT, _D), jnp.float32),
        scratch_types=[
            pltpu.VMEM((_BPW,), jnp.int32),
            pltpu.VMEM((_BPW, _D), jnp.float32),
            pltpu.SemaphoreType.DMA,
        ],
    )
    def k(table_hbm, idx_hbm, out_hbm, idx_v, rows_v, sem):
        wid = lax.axis_index("s") * 2 + lax.axis_index("c")
        base = wid * _BPW
        pltpu.sync_copy(idx_hbm.at[pl.ds(base, _BPW)], idx_v)
        pltpu.async_copy(table_hbm.at[idx_v], rows_v, sem).wait()
        pltpu.sync_copy(rows_v, out_hbm.at[pl.ds(base, _BPW)])

    return k(table, fidx)


def kernel(x, y, Wc, bc, Wr, br, Wl1, bl1, Wl2, bl2, Wbe, bbe, Wknn, bknn,
           Wg1, bg1, Wg2, bg2, Wd1, bd1, Wd2, bd2, W2l1, b2l1, W2l2, b2l2,
           Wlin, blin):
    w0 = _scores(x, y, Wc, bc, Wr, br, Wl1, bl1, Wl2, bl2, Wbe, bbe,
                 Wknn, bknn, Wg1, bg1, Wg2, bg2, Wd1, bd1, Wd2, bd2,
                 W2l1, b2l1, W2l2, b2l2, Wlin, blin)

    idx_sorted = _topk_indices(w0)
    idx = idx_sorted[:, :NDS]

    fidx = (idx + jnp.arange(B, dtype=jnp.int32)[:, None] * N).reshape(-1)
    fidx = jnp.concatenate(
        [fidx, jnp.zeros((_NOUT - B * NDS,), jnp.int32)])

    table = jnp.concatenate(
        [x[:, 0, :, :], y[:, :, None],
         jnp.zeros((B, N, _D - 5), jnp.float32)], axis=2).reshape(B * N, _D)

    rows = _sc_gather(table, fidx)[:B * NDS]
    x_ds = rows[:, :4].reshape(B, NDS, 4)[:, None, :, :]
    y_ds = rows[:, 4].reshape(B, NDS)
    return x_ds, y_ds, w0


# speedup vs baseline: 7.9066x; 7.9066x over previous
import functools

import jax
import jax.numpy as jnp
from jax import lax
from jax.experimental import pallas as pl
from jax.experimental.pallas import tpu as pltpu
from jax.experimental.pallas import tpu_sc as plsc

B, N, C, K = 8, 2000, 128, 9
NDS = 1000
EPS = 1e-5
NPAD = 2048


def _bn3(x):
    m = jnp.mean(x, axis=(0, 2), keepdims=True)
    v = jnp.var(x, axis=(0, 2), keepdims=True)
    return (x - m) / jnp.sqrt(v + EPS)


def _bn4(x):
    m = jnp.mean(x, axis=(0, 2, 3), keepdims=True)
    v = jnp.var(x, axis=(0, 2, 3), keepdims=True)
    return (x - m) / jnp.sqrt(v + EPS)


def _bn1(x):
    m = jnp.mean(x, axis=0, keepdims=True)
    v = jnp.var(x, axis=0, keepdims=True)
    return (x - m) / jnp.sqrt(v + EPS)


def _in3(x):
    m = jnp.mean(x, axis=2, keepdims=True)
    v = jnp.var(x, axis=2, keepdims=True)
    return (x - m) / jnp.sqrt(v + EPS)


def _c1(x, W, b):
    return jnp.einsum('oc,bcn->bon', W, x) + b[None, :, None]


def _c2(x, W, b, sw):
    out = lax.conv_general_dilated(
        x, W, window_strides=(1, sw), padding='VALID',
        dimension_numbers=('NCHW', 'OIHW', 'NCHW'))
    return out + b[None, :, None, None]


def _resnet(x, Wr, br, W1, b1, W2, b2, pre):
    x1 = _c1(x, Wr, br) if pre else x
    out = jax.nn.relu(_bn3(_in3(_c1(x, W1, b1))))
    out = _bn3(_in3(_c1(out, W2, b2)))
    return jax.nn.relu(out + x1)


_RT = 200


def _knn_body(pd_ref, idx_ref):
    v = pd_ref[0]
    colio = lax.broadcasted_iota(jnp.int32, (_RT, N), 1)
    for k in range(K):
        m = jnp.max(v, axis=1, keepdims=True)
        sel = jnp.min(jnp.where(v == m, colio, N), axis=1)
        idx_ref[0, :, k] = sel
        v = jnp.where(colio == sel[:, None], -jnp.inf, v)


def _knn_topk(pd):
    return pl.pallas_call(
        _knn_body,
        grid=(B, N // _RT),
        in_specs=[pl.BlockSpec((1, _RT, N), lambda b, t: (b, t, 0))],
        out_specs=pl.BlockSpec((1, _RT, 16), lambda b, t: (b, t, 0)),
        out_shape=jax.ShapeDtypeStruct((B, N, 16), jnp.int32),
    )(pd)


def _graph_feature(x, k):
    inner = -2.0 * jnp.einsum('bcn,bcm->bnm', x, x)
    xx = jnp.sum(x * x, axis=1)
    pd = -xx[:, :, None] - inner - xx[:, None, :]
    idx = _knn_topk(pd)[:, :, :K]
    xt = jnp.transpose(x, (0, 2, 1))
    fidx = (idx + jnp.arange(B, dtype=jnp.int32)[:, None, None] * N).reshape(-1)
    fidx = jnp.concatenate(
        [fidx, jnp.zeros((_GF_NOUT - B * N * K,), jnp.int32)])
    rows = _sc_gather_wide(xt.reshape(B * N, C), fidx,
                           _GF_NOUT, _GF_CHUNKS)[:B * N * K]
    feat = rows.reshape(B, N, K, C)
    xrep = jnp.broadcast_to(xt[:, :, None, :], feat.shape)
    f = jnp.concatenate([xrep, xrep - feat], axis=3)
    return jnp.transpose(f, (0, 3, 1, 2))


def _scores(x, y, Wc, bc, Wr, br, Wl1, bl1, Wl2, bl2, Wbe, bbe, Wknn, bknn,
            Wg1, bg1, Wg2, bg2, Wd1, bd1, Wd2, bd2, W2l1, b2l1, W2l2, b2l2,
            Wlin, blin):
    xt = jnp.transpose(x[:, 0, :, :], (0, 2, 1))
    out0 = jax.nn.relu(_bn3(_c1(xt, Wc, bc)))
    d = _resnet(out0, Wr, br, Wl1, bl1, Wl2, bl2, True)
    x1 = jax.nn.relu(_bn3(_c1(d, Wbe, bbe)))
    xl = jax.nn.relu(_bn3(_c1(x1, Wknn, bknn)))
    f = _graph_feature(xl, K)
    h = jax.nn.relu(_bn4(_c2(f, Wd1, bd1, 3)))
    h = jax.nn.relu(_bn4(_c2(h, Wd2, bd2, 1)))
    gv = jnp.mean(x1, axis=2)
    g1 = jax.nn.relu(_bn1(gv @ Wg1.T + bg1))
    g2 = _bn1(g1 @ Wg2.T + bg2)
    xlg = g2[:, :, None, None] + h
    wei = jax.nn.sigmoid(xlg)[:, :, :, 0]
    out = 2.0 * d * wei + 2.0 * out0 * (1.0 - wei)
    out = _resnet(out, Wr, br, W2l1, b2l1, W2l2, b2l2, False)
    return _c1(out, Wlin, blin)[:, 0, :]


def _xor_swap(a, j):
    lower = (lax.broadcasted_iota(jnp.int32, a.shape, 1) & j) == 0
    return jnp.where(lower, jnp.roll(a, -j, axis=1), jnp.roll(a, j, axis=1))


def _topk_body(w_ref, idx_ref):
    v = w_ref[...]
    iota = lax.broadcasted_iota(jnp.int32, (B, NPAD), 1)
    idx = iota
    k = 2
    while k <= NPAD:
        j = k // 2
        while j >= 1:
            pv = _xor_swap(v, j)
            pidx = _xor_swap(idx, j)
            desc = (iota & k) == 0
            lower = (iota & j) == 0
            before = (v > pv) | ((v == pv) & (idx < pidx))
            take_self = desc == (lower == before)
            v = jnp.where(take_self, v, pv)
            idx = jnp.where(take_self, idx, pidx)
            j //= 2
        k *= 2
    idx_ref[...] = idx


def _topk_indices(w0):
    wp = jnp.concatenate(
        [w0, jnp.full((B, NPAD - N), -jnp.inf, jnp.float32)], axis=1)
    return pl.pallas_call(
        _topk_body,
        out_shape=jax.ShapeDtypeStruct((B, NPAD), jnp.int32),
    )(wp)


_D = 128
_NW = 32

_DS_NOUT = 8192
_GF_NOUT = 147456
_GF_CHUNKS = 8


def _sc_gather_wide(table, fidx, nout, nchunks):
    bpw = nout // _NW
    chunk = bpw // nchunks
    mesh = plsc.VectorSubcoreMesh(core_axis_name="c", subcore_axis_name="s")

    @functools.partial(
        pl.kernel, mesh=mesh,
        out_type=jax.ShapeDtypeStruct((nout, _D), jnp.float32),
        scratch_types=[
            pltpu.VMEM((chunk,), jnp.int32),
            pltpu.VMEM((chunk, _D), jnp.float32),
            pltpu.SemaphoreType.DMA,
        ],
    )
    def k(table_hbm, idx_hbm, out_hbm, idx_v, rows_v, sem):
        wid = lax.axis_index("s") * 2 + lax.axis_index("c")
        for c in range(nchunks):
            base = wid * bpw + c * chunk
            pltpu.sync_copy(idx_hbm.at[pl.ds(base, chunk)], idx_v)
            pltpu.async_copy(table_hbm.at[idx_v], rows_v, sem).wait()
            pltpu.sync_copy(rows_v, out_hbm.at[pl.ds(base, chunk)])

    return k(table, fidx)


def kernel(x, y, Wc, bc, Wr, br, Wl1, bl1, Wl2, bl2, Wbe, bbe, Wknn, bknn,
           Wg1, bg1, Wg2, bg2, Wd1, bd1, Wd2, bd2, W2l1, b2l1, W2l2, b2l2,
           Wlin, blin):
    w0 = _scores(x, y, Wc, bc, Wr, br, Wl1, bl1, Wl2, bl2, Wbe, bbe,
                 Wknn, bknn, Wg1, bg1, Wg2, bg2, Wd1, bd1, Wd2, bd2,
                 W2l1, b2l1, W2l2, b2l2, Wlin, blin)

    idx_sorted = _topk_indices(w0)
    idx = idx_sorted[:, :NDS]

    fidx = (idx + jnp.arange(B, dtype=jnp.int32)[:, None] * N).reshape(-1)
    fidx = jnp.concatenate(
        [fidx, jnp.zeros((_DS_NOUT - B * NDS,), jnp.int32)])

    table = jnp.concatenate(
        [x[:, 0, :, :], y[:, :, None],
         jnp.zeros((B, N, _D - 5), jnp.float32)], axis=2).reshape(B * N, _D)

    rows = _sc_gather_wide(table, fidx, _DS_NOUT, 1)[:B * NDS]
    x_ds = rows[:, :4].reshape(B, NDS, 4)[:, None, :, :]
    y_ds = rows[:, 4].reshape(B, NDS)
    return x_ds, y_ds, w0


# speedup vs baseline: 8.1041x; 1.0250x over previous
import functools

import jax
import jax.numpy as jnp
from jax import lax
from jax.experimental import pallas as pl
from jax.experimental.pallas import tpu as pltpu
from jax.experimental.pallas import tpu_sc as plsc

B, N, C, K = 8, 2000, 128, 9
NDS = 1000
EPS = 1e-5
NPAD = 2048


def _bn3(x):
    m = jnp.mean(x, axis=(0, 2), keepdims=True)
    v = jnp.var(x, axis=(0, 2), keepdims=True)
    return (x - m) / jnp.sqrt(v + EPS)


def _bn4(x):
    m = jnp.mean(x, axis=(0, 2, 3), keepdims=True)
    v = jnp.var(x, axis=(0, 2, 3), keepdims=True)
    return (x - m) / jnp.sqrt(v + EPS)


def _bn1(x):
    m = jnp.mean(x, axis=0, keepdims=True)
    v = jnp.var(x, axis=0, keepdims=True)
    return (x - m) / jnp.sqrt(v + EPS)


def _in3(x):
    m = jnp.mean(x, axis=2, keepdims=True)
    v = jnp.var(x, axis=2, keepdims=True)
    return (x - m) / jnp.sqrt(v + EPS)


def _c1(x, W, b):
    return jnp.einsum('oc,bcn->bon', W, x) + b[None, :, None]


def _c2(x, W, b, sw):
    out = lax.conv_general_dilated(
        x, W, window_strides=(1, sw), padding='VALID',
        dimension_numbers=('NCHW', 'OIHW', 'NCHW'))
    return out + b[None, :, None, None]


def _resnet(x, Wr, br, W1, b1, W2, b2, pre):
    x1 = _c1(x, Wr, br) if pre else x
    out = jax.nn.relu(_bn3(_in3(_c1(x, W1, b1))))
    out = _bn3(_in3(_c1(out, W2, b2)))
    return jax.nn.relu(out + x1)


_RT = 200


def _knn_body(pd_ref, idx_ref):
    v = pd_ref[0]
    colio = lax.broadcasted_iota(jnp.int32, (_RT, N), 1)
    for k in range(K):
        m = jnp.max(v, axis=1, keepdims=True)
        sel = jnp.min(jnp.where(v == m, colio, N), axis=1)
        idx_ref[0, :, k] = sel
        v = jnp.where(colio == sel[:, None], -jnp.inf, v)


def _knn_topk(pd):
    return pl.pallas_call(
        _knn_body,
        grid=(B, N // _RT),
        in_specs=[pl.BlockSpec((1, _RT, N), lambda b, t: (b, t, 0))],
        out_specs=pl.BlockSpec((1, _RT, 16), lambda b, t: (b, t, 0)),
        out_shape=jax.ShapeDtypeStruct((B, N, 16), jnp.int32),
    )(pd)


def _graph_feature(x, k):
    inner = -2.0 * jnp.einsum('bcn,bcm->bnm', x, x)
    xx = jnp.sum(x * x, axis=1)
    pd = -xx[:, :, None] - inner - xx[:, None, :]
    idx = _knn_topk(pd)[:, :, :K]
    xt = jnp.transpose(x, (0, 2, 1))
    fidx = (idx + jnp.arange(B, dtype=jnp.int32)[:, None, None] * N).reshape(-1)
    fidx = jnp.concatenate(
        [fidx, jnp.zeros((_GF_NOUT - B * N * K,), jnp.int32)])
    rows = _sc_gather_wide(xt.reshape(B * N, C), fidx,
                           _GF_NOUT, _GF_CHUNKS)[:B * N * K]
    feat = rows.reshape(B, N, K, C)
    xrep = jnp.broadcast_to(xt[:, :, None, :], feat.shape)
    f = jnp.concatenate([xrep, xrep - feat], axis=3)
    return jnp.transpose(f, (0, 3, 1, 2))


def _scores(x, y, Wc, bc, Wr, br, Wl1, bl1, Wl2, bl2, Wbe, bbe, Wknn, bknn,
            Wg1, bg1, Wg2, bg2, Wd1, bd1, Wd2, bd2, W2l1, b2l1, W2l2, b2l2,
            Wlin, blin):
    xt = jnp.transpose(x[:, 0, :, :], (0, 2, 1))
    out0 = jax.nn.relu(_bn3(_c1(xt, Wc, bc)))
    d = _resnet(out0, Wr, br, Wl1, bl1, Wl2, bl2, True)
    x1 = jax.nn.relu(_bn3(_c1(d, Wbe, bbe)))
    xl = jax.nn.relu(_bn3(_c1(x1, Wknn, bknn)))
    f = _graph_feature(xl, K)
    h = jax.nn.relu(_bn4(_c2(f, Wd1, bd1, 3)))
    h = jax.nn.relu(_bn4(_c2(h, Wd2, bd2, 1)))
    gv = jnp.mean(x1, axis=2)
    g1 = jax.nn.relu(_bn1(gv @ Wg1.T + bg1))
    g2 = _bn1(g1 @ Wg2.T + bg2)
    xlg = g2[:, :, None, None] + h
    wei = jax.nn.sigmoid(xlg)[:, :, :, 0]
    out = 2.0 * d * wei + 2.0 * out0 * (1.0 - wei)
    out = _resnet(out, Wr, br, W2l1, b2l1, W2l2, b2l2, False)
    return _c1(out, Wlin, blin)[:, 0, :]


def _xor_swap(a, j):
    lower = (lax.broadcasted_iota(jnp.int32, a.shape, 1) & j) == 0
    return jnp.where(lower, jnp.roll(a, -j, axis=1), jnp.roll(a, j, axis=1))


def _topk_body(w_ref, idx_ref):
    v = w_ref[...]
    iota = lax.broadcasted_iota(jnp.int32, (B, NPAD), 1)
    idx = iota
    k = 2
    while k <= NPAD:
        j = k // 2
        while j >= 1:
            pv = _xor_swap(v, j)
            pidx = _xor_swap(idx, j)
            desc = (iota & k) == 0
            lower = (iota & j) == 0
            before = (v > pv) | ((v == pv) & (idx < pidx))
            take_self = desc == (lower == before)
            v = jnp.where(take_self, v, pv)
            idx = jnp.where(take_self, idx, pidx)
            j //= 2
        k *= 2
    idx_ref[...] = idx


def _topk_indices(w0):
    wp = jnp.concatenate(
        [w0, jnp.full((B, NPAD - N), -jnp.inf, jnp.float32)], axis=1)
    return pl.pallas_call(
        _topk_body,
        out_shape=jax.ShapeDtypeStruct((B, NPAD), jnp.int32),
    )(wp)


_D = 128
_NW = 32

_DS_NOUT = 8192
_GF_NOUT = 147456
_GF_CHUNKS = 16


def _sc_gather_wide(table, fidx, nout, nchunks):
    bpw = nout // _NW
    chunk = bpw // nchunks
    mesh = plsc.VectorSubcoreMesh(core_axis_name="c", subcore_axis_name="s")

    @functools.partial(
        pl.kernel, mesh=mesh,
        out_type=jax.ShapeDtypeStruct((nout, _D), jnp.float32),
        scratch_types=[
            pltpu.VMEM((chunk,), jnp.int32),
            pltpu.VMEM((chunk,), jnp.int32),
            pltpu.VMEM((chunk, _D), jnp.float32),
            pltpu.VMEM((chunk, _D), jnp.float32),
            pltpu.SemaphoreType.DMA,
            pltpu.SemaphoreType.DMA,
        ],
    )
    def k(table_hbm, idx_hbm, out_hbm, idx_v0, idx_v1, rows_v0, rows_v1,
          sem0, sem1):
        wid = lax.axis_index("s") * 2 + lax.axis_index("c")
        idx_bufs = (idx_v0, idx_v1)
        row_bufs = (rows_v0, rows_v1)
        sems = (sem0, sem1)

        def start(c):
            s = c % 2
            base = wid * bpw + c * chunk
            pltpu.sync_copy(idx_hbm.at[pl.ds(base, chunk)], idx_bufs[s])
            return pltpu.async_copy(table_hbm.at[idx_bufs[s]],
                                    row_bufs[s], sems[s])

        h = start(0)
        for c in range(nchunks):
            h_next = start(c + 1) if c + 1 < nchunks else None
            h.wait()
            base = wid * bpw + c * chunk
            pltpu.sync_copy(row_bufs[c % 2], out_hbm.at[pl.ds(base, chunk)])
            h = h_next

    return k(table, fidx)


def kernel(x, y, Wc, bc, Wr, br, Wl1, bl1, Wl2, bl2, Wbe, bbe, Wknn, bknn,
           Wg1, bg1, Wg2, bg2, Wd1, bd1, Wd2, bd2, W2l1, b2l1, W2l2, b2l2,
           Wlin, blin):
    w0 = _scores(x, y, Wc, bc, Wr, br, Wl1, bl1, Wl2, bl2, Wbe, bbe,
                 Wknn, bknn, Wg1, bg1, Wg2, bg2, Wd1, bd1, Wd2, bd2,
                 W2l1, b2l1, W2l2, b2l2, Wlin, blin)

    idx_sorted = _topk_indices(w0)
    idx = idx_sorted[:, :NDS]

    fidx = (idx + jnp.arange(B, dtype=jnp.int32)[:, None] * N).reshape(-1)
    fidx = jnp.concatenate(
        [fidx, jnp.zeros((_DS_NOUT - B * NDS,), jnp.int32)])

    table = jnp.concatenate(
        [x[:, 0, :, :], y[:, :, None],
         jnp.zeros((B, N, _D - 5), jnp.float32)], axis=2).reshape(B * N, _D)

    rows = _sc_gather_wide(table, fidx, _DS_NOUT, 1)[:B * NDS]
    x_ds = rows[:, :4].reshape(B, NDS, 4)[:, None, :, :]
    y_ds = rows[:, 4].reshape(B, NDS)
    return x_ds, y_ds, w0
